# baseline (speedup 1.0000x reference)
import functools

import jax
import jax.numpy as jnp
from jax import lax
from jax.experimental import pallas as pl
from jax.experimental.pallas import tpu as pltpu
from jax.experimental.pallas import tpu_sc as plsc

_VOCAB = 100000
_EMB = 128
_B = 4096
_L = 50

_NC = 2
_NS = 16
_NW = _NC * _NS
_N = _B * _L
_PER_W = _N // _NW
_CHUNK = 128
_NCHUNK = _PER_W // _CHUNK
_NBUF = 4

_mesh = plsc.VectorSubcoreMesh(core_axis_name="c", subcore_axis_name="s")


@functools.partial(
    pl.kernel,
    out_type=jax.ShapeDtypeStruct((_N, _EMB), jnp.float32),
    mesh=_mesh,
    scratch_types=[
        pltpu.VMEM((_NCHUNK, _CHUNK), jnp.int32),
        pltpu.VMEM((_NBUF, _CHUNK, _EMB), jnp.float32),
        pltpu.SemaphoreType.DMA,
        pltpu.SemaphoreType.DMA,
        pltpu.SemaphoreType.DMA,
        pltpu.SemaphoreType.DMA,
        pltpu.SemaphoreType.DMA,
        pltpu.SemaphoreType.DMA,
        pltpu.SemaphoreType.DMA,
        pltpu.SemaphoreType.DMA,
    ],
)
def _emb_lookup(idx_hbm, table_hbm, out_hbm, idx_v, rows_v,
                g0, g1, g2, g3, s0, s1, s2, s3):
    wid = lax.axis_index("s") * _NC + lax.axis_index("c")
    base = wid * _PER_W
    gsem = (g0, g1, g2, g3)
    ssem = (s0, s1, s2, s3)
    pltpu.sync_copy(idx_hbm.at[wid], idx_v)

    def start_gather(j, b):
        pltpu.make_async_copy(table_hbm.at[idx_v.at[j]], rows_v.at[b], gsem[b]).start()

    def wait_gather(b):
        pltpu.make_async_copy(table_hbm.at[pl.ds(0, _CHUNK)], rows_v.at[b], gsem[b]).wait()

    def store(j, b):
        return pltpu.make_async_copy(
            rows_v.at[b], out_hbm.at[pl.ds(base + j * _CHUNK, _CHUNK)], ssem[b]
        )

    for b in range(_NBUF):
        start_gather(b, b)

    def body(i, carry):
        j0 = i * _NBUF
        for b in range(_NBUF):
            j = j0 + b
            wait_gather(b)
            store(j, b).start()
            store(j, b).wait()
            start_gather(j + _NBUF, b)
        return carry

    n_main = ((_NCHUNK - _NBUF) // _NBUF) * _NBUF
    lax.fori_loop(0, n_main // _NBUF, body, 0)

    for j in range(n_main, _NCHUNK):
        b = j % _NBUF
        wait_gather(b)
        pltpu.sync_copy(rows_v.at[b], out_hbm.at[pl.ds(base + j * _CHUNK, _CHUNK)])
        if j + _NBUF < _NCHUNK:
            start_gather(j + _NBUF, b)


def kernel(input, table):
    idx = input.reshape(_NW, _NCHUNK, _CHUNK).astype(jnp.int32)
    out = _emb_lookup(idx, table)
    return out.reshape(_B, _L, _EMB)


# speedup vs baseline: 1.7844x; 1.7844x over previous
import functools

import jax
import jax.numpy as jnp
from jax import lax
from jax.experimental import pallas as pl
from jax.experimental.pallas import tpu as pltpu
from jax.experimental.pallas import tpu_sc as plsc

_VOCAB = 100000
_EMB = 128
_B = 4096
_L = 50

_NC = 2
_NS = 16
_NW = _NC * _NS
_ROWS_W = _B // _NW
_NBUF = 4

_mesh = plsc.VectorSubcoreMesh(core_axis_name="c", subcore_axis_name="s")


@functools.partial(
    pl.kernel,
    out_type=jax.ShapeDtypeStruct((_B, _L, _EMB), jnp.float32),
    mesh=_mesh,
    scratch_types=[
        pltpu.VMEM((_ROWS_W, _L), jnp.int32),
        pltpu.VMEM((_L, _EMB), jnp.float32),
        pltpu.VMEM((_L, _EMB), jnp.float32),
        pltpu.VMEM((_L, _EMB), jnp.float32),
        pltpu.VMEM((_L, _EMB), jnp.float32),
        pltpu.SemaphoreType.DMA,
        pltpu.SemaphoreType.DMA,
        pltpu.SemaphoreType.DMA,
        pltpu.SemaphoreType.DMA,
        pltpu.SemaphoreType.DMA,
        pltpu.SemaphoreType.DMA,
        pltpu.SemaphoreType.DMA,
        pltpu.SemaphoreType.DMA,
    ],
)
def _emb_lookup(idx_hbm, table_hbm, out_hbm, idx_v, r0, r1, r2, r3,
                g0, g1, g2, g3, s0, s1, s2, s3):
    wid = lax.axis_index("s") * _NC + lax.axis_index("c")
    base = wid * _ROWS_W
    rows_v = (r0, r1, r2, r3)
    gsem = (g0, g1, g2, g3)
    ssem = (s0, s1, s2, s3)
    pltpu.sync_copy(idx_hbm.at[pl.ds(base, _ROWS_W)], idx_v)

    def start_gather(j, b):
        pltpu.make_async_copy(table_hbm.at[idx_v.at[j]], rows_v[b], gsem[b]).start()

    def wait_gather(j, b):
        pltpu.make_async_copy(table_hbm.at[idx_v.at[j]], rows_v[b], gsem[b]).wait()

    def store(j, b):
        return pltpu.make_async_copy(rows_v[b], out_hbm.at[base + j], ssem[b])

    for b in range(_NBUF):
        start_gather(b, b)

    def body(i, carry):
        j0 = i * _NBUF
        for b in range(_NBUF):
            j = j0 + b
            wait_gather(j, b)
            store(j, b).start()
            store(j, b).wait()
            start_gather(j + _NBUF, b)
        return carry

    n_main = ((_ROWS_W - _NBUF) // _NBUF) * _NBUF
    lax.fori_loop(0, n_main // _NBUF, body, 0)

    for j in range(n_main, _ROWS_W):
        b = j % _NBUF
        wait_gather(j, b)
        pltpu.sync_copy(rows_v[b], out_hbm.at[base + j])
        if j + _NBUF < _ROWS_W:
            start_gather(j + _NBUF, b)


def kernel(input, table):
    return _emb_lookup(input, table)


# speedup vs baseline: 3.2042x; 1.7957x over previous
import functools

import jax
import jax.numpy as jnp
from jax import lax
from jax.experimental import pallas as pl
from jax.experimental.pallas import tpu as pltpu
from jax.experimental.pallas import tpu_sc as plsc

_VOCAB = 100000
_EMB = 128
_B = 4096
_L = 50

_NC = 2
_NS = 16
_NW = _NC * _NS
_BW = _B // _NW
_NBUF = 4

_mesh = plsc.VectorSubcoreMesh(core_axis_name="c", subcore_axis_name="s")


@functools.partial(
    pl.kernel,
    out_type=jax.ShapeDtypeStruct((_L, _B, _EMB), jnp.float32),
    mesh=_mesh,
    scratch_types=[
        pltpu.VMEM((_L, _BW), jnp.int32),
        pltpu.VMEM((_BW, _EMB), jnp.float32),
        pltpu.VMEM((_BW, _EMB), jnp.float32),
        pltpu.VMEM((_BW, _EMB), jnp.float32),
        pltpu.VMEM((_BW, _EMB), jnp.float32),
        pltpu.SemaphoreType.DMA,
        pltpu.SemaphoreType.DMA,
        pltpu.SemaphoreType.DMA,
        pltpu.SemaphoreType.DMA,
        pltpu.SemaphoreType.DMA,
        pltpu.SemaphoreType.DMA,
        pltpu.SemaphoreType.DMA,
        pltpu.SemaphoreType.DMA,
    ],
)
def _emb_lookup(idx_hbm, table_hbm, out_hbm, idx_v, r0, r1, r2, r3,
                g0, g1, g2, g3, s0, s1, s2, s3):
    wid = lax.axis_index("s") * _NC + lax.axis_index("c")
    base = wid * _BW
    rows_v = (r0, r1, r2, r3)
    gsem = (g0, g1, g2, g3)
    ssem = (s0, s1, s2, s3)
    pltpu.sync_copy(idx_hbm.at[:, pl.ds(base, _BW)], idx_v)

    def start_gather(l, b):
        pltpu.make_async_copy(table_hbm.at[idx_v.at[l]], rows_v[b], gsem[b]).start()

    def wait_gather(l, b):
        pltpu.make_async_copy(table_hbm.at[idx_v.at[l]], rows_v[b], gsem[b]).wait()

    def store(l, b):
        return pltpu.make_async_copy(
            rows_v[b], out_hbm.at[l, pl.ds(base, _BW)], ssem[b]
        )

    for b in range(_NBUF):
        start_gather(b, b)

    def body(i, carry):
        l0 = i * _NBUF
        for b in range(_NBUF):
            l = l0 + b
            wait_gather(l, b)
            store(l, b).start()
            store(l, b).wait()
            start_gather(l + _NBUF, b)
        return carry

    n_main = ((_L - _NBUF) // _NBUF) * _NBUF
    lax.fori_loop(0, n_main // _NBUF, body, 0)

    for l in range(n_main, _L):
        b = l % _NBUF
        wait_gather(l, b)
        pltpu.sync_copy(rows_v[b], out_hbm.at[l, pl.ds(base, _BW)])
        if l + _NBUF < _L:
            start_gather(l + _NBUF, b)


def kernel(input, table):
    out_t = _emb_lookup(input.T, table)
    return out_t.transpose(1, 0, 2)


# speedup vs baseline: 3.2090x; 1.0015x over previous
import functools

import jax
import jax.numpy as jnp
from jax import lax
from jax.experimental import pallas as pl
from jax.experimental.pallas import tpu as pltpu
from jax.experimental.pallas import tpu_sc as plsc

_VOCAB = 100000
_EMB = 128
_B = 4096
_L = 50

_NC = 2
_NS = 16
_NW = _NC * _NS
_BW = _B // _NW
_NBUF = 4

_mesh = plsc.VectorSubcoreMesh(core_axis_name="c", subcore_axis_name="s")


@functools.partial(
    pl.kernel,
    out_type=jax.ShapeDtypeStruct((_L, _B, _EMB), jnp.float32),
    mesh=_mesh,
    scratch_types=[
        pltpu.VMEM((_L, _BW), jnp.int32),
        pltpu.VMEM((_BW, _EMB), jnp.float32),
        pltpu.VMEM((_BW, _EMB), jnp.float32),
        pltpu.VMEM((_BW, _EMB), jnp.float32),
        pltpu.VMEM((_BW, _EMB), jnp.float32),
        pltpu.SemaphoreType.DMA,
        pltpu.SemaphoreType.DMA,
        pltpu.SemaphoreType.DMA,
        pltpu.SemaphoreType.DMA,
        pltpu.SemaphoreType.DMA,
        pltpu.SemaphoreType.DMA,
        pltpu.SemaphoreType.DMA,
        pltpu.SemaphoreType.DMA,
    ],
)
def _emb_lookup(idx_hbm, table_hbm, out_hbm, idx_v, r0, r1, r2, r3,
                g0, g1, g2, g3, s0, s1, s2, s3):
    wid = lax.axis_index("s") * _NC + lax.axis_index("c")
    base = wid * _BW
    rows_v = (r0, r1, r2, r3)
    gsem = (g0, g1, g2, g3)
    ssem = (s0, s1, s2, s3)
    pltpu.sync_copy(idx_hbm.at[:, pl.ds(base, _BW)], idx_v)

    def start_gather(l, b):
        pltpu.make_async_copy(table_hbm.at[idx_v.at[l]], rows_v[b], gsem[b]).start()

    def wait_gather(l, b):
        pltpu.make_async_copy(table_hbm.at[idx_v.at[l]], rows_v[b], gsem[b]).wait()

    def store(l, b):
        return pltpu.make_async_copy(
            rows_v[b], out_hbm.at[l, pl.ds(base, _BW)], ssem[b]
        )

    AHEAD = 2

    def step(l, b, bg, do_wait_store, do_start_gather):
        wait_gather(l, b)
        store(l, b).start()
        if do_start_gather:
            if do_wait_store:
                store(l - AHEAD, bg).wait()
            start_gather(l + AHEAD, bg)

    for l in range(AHEAD):
        start_gather(l, l % _NBUF)
    for l in range(AHEAD):
        step(l, l % _NBUF, (l + AHEAD) % _NBUF, False, True)

    def body(i, carry):
        l0 = AHEAD + i * _NBUF
        for k in range(_NBUF):
            step(l0 + k, (AHEAD + k) % _NBUF, k, True, True)
        return carry

    n_main = ((_L - 2 * AHEAD - AHEAD) // _NBUF) * _NBUF
    lax.fori_loop(0, n_main // _NBUF, body, 0)

    for l in range(AHEAD + n_main, _L):
        step(l, l % _NBUF, (l + AHEAD) % _NBUF, True, l + AHEAD < _L)
    for l in range(_L - AHEAD - 2, _L):
        store(l, l % _NBUF).wait()


def kernel(input, table):
    out_t = _emb_lookup(input.T, table)
    return out_t.transpose(1, 0, 2)
